# baseline (speedup 1.0000x reference)
import functools

import jax
import jax.numpy as jnp
from jax import lax
from jax.experimental import pallas as pl
from jax.experimental.pallas import tpu as pltpu
from jax.experimental.pallas import tpu_sc as plsc

NC = 2
NS = 16
CHUNK = 128
F = 128


def _round_up(n, m):
    return ((n + m - 1) // m) * m


@functools.cache
def _sc_scatter(M, N_nodes, e_core):
    K = 3
    rows_sub = N_nodes // NS
    e_sub = e_core // NS
    chunks = e_sub // CHUNK
    assert N_nodes % (NS * 8) == 0 and e_core % (NS * CHUNK) == 0
    assert chunks % K == 0

    mesh = plsc.VectorSubcoreMesh(core_axis_name="c", subcore_axis_name="s",
                                  num_cores=NC, num_subcores=NS)

    @functools.partial(
        pl.kernel,
        out_type=jax.ShapeDtypeStruct((2 * N_nodes, F), jnp.float32),
        mesh=mesh,
        scratch_types=[
            pltpu.VMEM((K, 2, CHUNK), jnp.int32),
            pltpu.VMEM((K, CHUNK, F), jnp.float32),
            pltpu.VMEM_SHARED((N_nodes, F), jnp.float32),
            pltpu.SemaphoreType.DMA((K,)),
            pltpu.SemaphoreType.DMA((K,)),
        ],
    )
    def scatter_kernel(h_hbm, sd_hbm, init_hbm, out_hbm, sd, rw, acc, gs, ss):
        c = lax.axis_index("c")
        s = lax.axis_index("s")
        row0 = c * N_nodes + s * rows_sub
        pltpu.sync_copy(init_hbm.at[pl.ds(row0, rows_sub)],
                        acc.at[pl.ds(s * rows_sub, rows_sub)])
        plsc.subcore_barrier()

        cbase = (c * NS + s) * chunks

        def group(g, carry):
            base = cbase + g * K
            for k in range(K):
                pltpu.sync_copy(sd_hbm.at[base + k], sd.at[k])
                pltpu.async_copy(h_hbm.at[sd.at[k, 0]], rw.at[k],
                                 gs.at[k]).wait()
                pltpu.sync_copy(rw.at[k], acc.at[sd.at[k, 1]], add=True)
            return carry

        lax.fori_loop(0, chunks // K, group, 0)
        plsc.subcore_barrier()
        pltpu.sync_copy(acc.at[pl.ds(s * rows_sub, rows_sub)],
                        out_hbm.at[pl.ds(row0, rows_sub)])

    return scatter_kernel


@functools.cache
def _sc_gather(n_rows, B):
    K = 5
    b_per_w = B // (NC * NS)
    chunks = b_per_w // CHUNK
    assert B % (NC * NS * CHUNK) == 0 and chunks % K == 0

    mesh = plsc.VectorSubcoreMesh(core_axis_name="c", subcore_axis_name="s",
                                  num_cores=NC, num_subcores=NS)

    @functools.partial(
        pl.kernel,
        out_type=jax.ShapeDtypeStruct((B, F), jnp.float32),
        mesh=mesh,
        scratch_types=[
            pltpu.VMEM((K, CHUNK), jnp.int32),
            pltpu.VMEM((K, CHUNK, F), jnp.float32),
            pltpu.SemaphoreType.DMA((K,)),
            pltpu.SemaphoreType.DMA((K,)),
        ],
    )
    def gather_kernel(tab_hbm, idx_hbm, out_hbm, idv, rw, gs, os):
        wid = lax.axis_index("s") * NC + lax.axis_index("c")
        base = wid * b_per_w

        def group(g, carry):
            off = base + g * (K * CHUNK)
            for k in range(K):
                pltpu.sync_copy(idx_hbm.at[pl.ds(off + k * CHUNK, CHUNK)],
                                idv.at[k])
                pltpu.async_copy(tab_hbm.at[idv.at[k]], rw.at[k],
                                 gs.at[k]).wait()
                pltpu.sync_copy(rw.at[k],
                                out_hbm.at[pl.ds(off + k * CHUNK, CHUNK)])
            return carry

        lax.fori_loop(0, chunks // K, group, 0)

    return gather_kernel


def _tc1_body(p_ref, w_ref, b_ref, o_ref):
    a = p_ref[0] + p_ref[1]
    h = jnp.dot(a, w_ref[...], preferred_element_type=jnp.float32)
    h = jnp.maximum(h + b_ref[...], 0.0)
    o_ref[0] = h[:, :F]
    o_ref[1] = h[:, F:]


def _tc_layer1(P, W1, b1, n, R=2000):
    npad = P.shape[1]
    return pl.pallas_call(
        _tc1_body,
        grid=(n // R,),
        in_specs=[
            pl.BlockSpec((2, R, F), lambda i: (0, i, 0)),
            pl.BlockSpec(W1.shape, lambda i: (0, 0)),
            pl.BlockSpec((1, 2 * F), lambda i: (0, 0)),
        ],
        out_specs=pl.BlockSpec((2, R, F), lambda i: (0, i, 0)),
        out_shape=jax.ShapeDtypeStruct((2, npad, F), jnp.float32),
    )(P, W1, b1.reshape(1, -1))


def _tc2_body(a_ref, w2_ref, b2_ref, w3_ref, o_ref):
    w2 = w2_ref[...]
    h2 = (jnp.dot(a_ref[0], w2[:F], preferred_element_type=jnp.float32)
          + jnp.dot(a_ref[1], w2[F:], preferred_element_type=jnp.float32))
    h2 = jnp.maximum(h2 + b2_ref[...], 0.0)
    o_ref[...] = jnp.dot(h2, w3_ref[...], preferred_element_type=jnp.float32)


def _tc_layer23(A, W2, b2, W3, n, R=2000):
    npad = A.shape[1]
    return pl.pallas_call(
        _tc2_body,
        grid=(n // R,),
        in_specs=[
            pl.BlockSpec((2, R, F), lambda i: (0, i, 0)),
            pl.BlockSpec(W2.shape, lambda i: (0, 0)),
            pl.BlockSpec((1, 2 * F), lambda i: (0, 0)),
            pl.BlockSpec(W3.shape, lambda i: (0, 0)),
        ],
        out_specs=pl.BlockSpec((R, F), lambda i: (i, 0)),
        out_shape=jax.ShapeDtypeStruct((npad, F), jnp.float32),
    )(A, W2, b2.reshape(1, -1), W3)


def _tc3_body(q_ref, r_ref, w1_ref, b1_ref, w2_ref, b2_ref, w3_ref, b3_ref,
              o_ref):
    p = q_ref[...] * r_ref[...]
    z = jnp.dot(p, w1_ref[...], preferred_element_type=jnp.float32)
    z = jnp.maximum(z + b1_ref[...], 0.0)
    z = jnp.dot(z, w2_ref[...], preferred_element_type=jnp.float32)
    z = jnp.maximum(z + b2_ref[...], 0.0)
    z = jnp.dot(z, w3_ref[...], preferred_element_type=jnp.float32)
    o_ref[...] = jax.nn.sigmoid(z + b3_ref[...])


def _tc_mlp(G, PW1, Pb1, PW2, Pb2, PW3p, Pb3p, R=2048):
    t2 = G.shape[0]
    tp = t2 // 2
    nb = tp // R
    wcols = PW3p.shape[1]
    return pl.pallas_call(
        _tc3_body,
        grid=(nb,),
        in_specs=[
            pl.BlockSpec((R, F), lambda i: (i, 0)),
            pl.BlockSpec((R, F), lambda i: (i + nb, 0)),
            pl.BlockSpec(PW1.shape, lambda i: (0, 0)),
            pl.BlockSpec((1, 2 * F), lambda i: (0, 0)),
            pl.BlockSpec(PW2.shape, lambda i: (0, 0)),
            pl.BlockSpec((1, 2 * F), lambda i: (0, 0)),
            pl.BlockSpec(PW3p.shape, lambda i: (0, 0)),
            pl.BlockSpec((1, wcols), lambda i: (0, 0)),
        ],
        out_specs=pl.BlockSpec((R, wcols), lambda i: (i, 0)),
        out_shape=jax.ShapeDtypeStruct((tp, wcols), jnp.float32),
    )(G, G, PW1, Pb1.reshape(1, -1), PW2, Pb2.reshape(1, -1), PW3p,
      Pb3p.reshape(1, -1))


def kernel(x, adj_t, train_edges, W1, b1, W2, b2, W3, b3,
           PW1, Pb1, PW2, Pb2, PW3, Pb3):
    n, d_in = x.shape
    e = adj_t.shape[1]
    t = train_edges.shape[0]
    assert d_in == F
    n_pad = _round_up(n, NS * 8)

    src, dst = adj_t[0], adj_t[1]

    def _mk_sd(src_flat, dst_flat):
        sd = jnp.stack([src_flat.reshape(-1, CHUNK),
                        dst_flat.reshape(-1, CHUNK)], axis=1)
        return jnp.concatenate([sd, jnp.zeros((2, 2, CHUNK), jnp.int32)])

    ec = e // 2
    ec_pad = _round_up(ec, NS * CHUNK * 3)
    pe = ec_pad - ec
    src_es = jnp.concatenate(
        [src.reshape(2, ec), jnp.zeros((2, pe), jnp.int32)], axis=1).reshape(-1)
    dst_es = jnp.concatenate(
        [dst.reshape(2, ec), jnp.full((2, pe), n, jnp.int32)], axis=1).reshape(-1)
    sd_es = _mk_sd(src_es, dst_es)

    e_pad = _round_up(e, NS * CHUNK * 3)
    pf = e_pad - e
    src_p = jnp.concatenate([src, jnp.zeros((pf,), jnp.int32)])
    dst_p = jnp.concatenate([dst, jnp.full((pf,), n, jnp.int32)])
    src_fs = jnp.concatenate([src_p, src_p + n_pad])
    dst_fs = jnp.concatenate([dst_p, dst_p])
    sd_fs = _mk_sd(src_fs, dst_fs)

    zeros_init = jnp.zeros((2 * n_pad, F), jnp.float32)

    P1 = _sc_scatter(n, n_pad, ec_pad)(x, sd_es, zeros_init)
    h1 = _tc_layer1(P1.reshape(2, n_pad, F), W1, b1, n)

    A1 = _sc_scatter(2 * n_pad, n_pad, e_pad)(h1.reshape(2 * n_pad, F),
                                              sd_fs, zeros_init)
    m3 = _tc_layer23(A1.reshape(2, n_pad, F), W2, b2, W3, n)

    init3 = jnp.concatenate(
        [jnp.broadcast_to(b3, (n_pad, F)), jnp.zeros((n_pad, F), jnp.float32)])
    P3 = _sc_scatter(n, n_pad, ec_pad)(m3, sd_es, init3)
    P3 = P3.reshape(2, n_pad, F)
    emb = P3[0] + P3[1]

    t_pad = _round_up(t, NC * NS * CHUNK * 5 // 2)
    pt = t_pad - t
    se = jnp.concatenate([train_edges[:, 0], jnp.zeros((pt,), jnp.int32)])
    de = jnp.concatenate([train_edges[:, 1], jnp.zeros((pt,), jnp.int32)])
    gidx = jnp.concatenate([se, de, jnp.zeros((2 * CHUNK,), jnp.int32)])
    G = _sc_gather(n_pad, 2 * t_pad)(emb, gidx)

    PW3p = jnp.pad(PW3, ((0, 0), (0, 8 - PW3.shape[1])))
    Pb3p = jnp.pad(Pb3, (0, 8 - Pb3.shape[0]))
    p = _tc_mlp(G, PW1, Pb1, PW2, Pb2, PW3p, Pb3p)

    return p[:t, :PW3.shape[1]][None]


# speedup vs baseline: 1.3487x; 1.3487x over previous
import functools

import jax
import jax.numpy as jnp
from jax import lax
from jax.experimental import pallas as pl
from jax.experimental.pallas import tpu as pltpu
from jax.experimental.pallas import tpu_sc as plsc

NC = 2
NS = 16
CHUNK = 128
F = 128


def _round_up(n, m):
    return ((n + m - 1) // m) * m


@functools.cache
def _sc_scatter(M, N_nodes, e_core):
    rows_sub = N_nodes // NS
    e_sub = e_core // NS
    chunks = e_sub // CHUNK
    assert N_nodes % (NS * 8) == 0 and e_core % (NS * CHUNK) == 0

    mesh = plsc.VectorSubcoreMesh(core_axis_name="c", subcore_axis_name="s",
                                  num_cores=NC, num_subcores=NS)

    @functools.partial(
        pl.kernel,
        out_type=jax.ShapeDtypeStruct((2 * N_nodes, F), jnp.float32),
        mesh=mesh,
        scratch_types=[
            pltpu.VMEM((CHUNK,), jnp.int32),
            pltpu.VMEM((CHUNK,), jnp.int32),
            pltpu.VMEM((CHUNK, F), jnp.float32),
            pltpu.VMEM_SHARED((N_nodes, F), jnp.float32),
            pltpu.SemaphoreType.DMA,
        ],
    )
    def scatter_kernel(h_hbm, src_hbm, dst_hbm, init_hbm, out_hbm,
                       sidx, didx, rows, acc, sem):
        c = lax.axis_index("c")
        s = lax.axis_index("s")
        row0 = c * N_nodes + s * rows_sub
        pltpu.sync_copy(init_hbm.at[pl.ds(row0, rows_sub)],
                        acc.at[pl.ds(s * rows_sub, rows_sub)])
        plsc.subcore_barrier()

        ebase = c * e_core + s * e_sub

        def body(i, carry):
            off = ebase + i * CHUNK
            pltpu.sync_copy(src_hbm.at[pl.ds(off, CHUNK)], sidx)
            pltpu.sync_copy(dst_hbm.at[pl.ds(off, CHUNK)], didx)
            pltpu.async_copy(h_hbm.at[sidx], rows, sem).wait()
            pltpu.sync_copy(rows, acc.at[didx], add=True)
            return carry

        lax.fori_loop(0, chunks, body, 0)
        plsc.subcore_barrier()
        pltpu.sync_copy(acc.at[pl.ds(s * rows_sub, rows_sub)],
                        out_hbm.at[pl.ds(row0, rows_sub)])

    return scatter_kernel


@functools.cache
def _sc_gather(n_rows, B):
    b_per_w = B // (NC * NS)
    chunks = b_per_w // CHUNK
    assert B % (NC * NS * CHUNK) == 0

    mesh = plsc.VectorSubcoreMesh(core_axis_name="c", subcore_axis_name="s",
                                  num_cores=NC, num_subcores=NS)

    @functools.partial(
        pl.kernel,
        out_type=jax.ShapeDtypeStruct((B, F), jnp.float32),
        mesh=mesh,
        scratch_types=[
            pltpu.VMEM((CHUNK,), jnp.int32),
            pltpu.VMEM((CHUNK, F), jnp.float32),
            pltpu.SemaphoreType.DMA,
        ],
    )
    def gather_kernel(tab_hbm, idx_hbm, out_hbm, idxv, rows, sem):
        wid = lax.axis_index("s") * NC + lax.axis_index("c")
        base = wid * b_per_w

        def body(i, carry):
            off = base + i * CHUNK
            pltpu.sync_copy(idx_hbm.at[pl.ds(off, CHUNK)], idxv)
            pltpu.async_copy(tab_hbm.at[idxv], rows, sem).wait()
            pltpu.sync_copy(rows, out_hbm.at[pl.ds(off, CHUNK)])
            return carry

        lax.fori_loop(0, chunks, body, 0)

    return gather_kernel


def _tc1_body(p_ref, w_ref, b_ref, o_ref):
    a = p_ref[0] + p_ref[1]
    h = jnp.dot(a, w_ref[...], preferred_element_type=jnp.float32)
    h = jnp.maximum(h + b_ref[...], 0.0)
    o_ref[0] = h[:, :F]
    o_ref[1] = h[:, F:]


def _tc_layer1(P, W1, b1, n, R=2000):
    npad = P.shape[1]
    return pl.pallas_call(
        _tc1_body,
        grid=(n // R,),
        in_specs=[
            pl.BlockSpec((2, R, F), lambda i: (0, i, 0)),
            pl.BlockSpec(W1.shape, lambda i: (0, 0)),
            pl.BlockSpec((1, 2 * F), lambda i: (0, 0)),
        ],
        out_specs=pl.BlockSpec((2, R, F), lambda i: (0, i, 0)),
        out_shape=jax.ShapeDtypeStruct((2, npad, F), jnp.float32),
    )(P, W1, b1.reshape(1, -1))


def _tc2_body(a_ref, w2_ref, b2_ref, w3_ref, o_ref):
    w2 = w2_ref[...]
    h2 = (jnp.dot(a_ref[0], w2[:F], preferred_element_type=jnp.float32)
          + jnp.dot(a_ref[1], w2[F:], preferred_element_type=jnp.float32))
    h2 = jnp.maximum(h2 + b2_ref[...], 0.0)
    o_ref[...] = jnp.dot(h2, w3_ref[...], preferred_element_type=jnp.float32)


def _tc_layer23(A, W2, b2, W3, n, R=2000):
    npad = A.shape[1]
    return pl.pallas_call(
        _tc2_body,
        grid=(n // R,),
        in_specs=[
            pl.BlockSpec((2, R, F), lambda i: (0, i, 0)),
            pl.BlockSpec(W2.shape, lambda i: (0, 0)),
            pl.BlockSpec((1, 2 * F), lambda i: (0, 0)),
            pl.BlockSpec(W3.shape, lambda i: (0, 0)),
        ],
        out_specs=pl.BlockSpec((R, F), lambda i: (i, 0)),
        out_shape=jax.ShapeDtypeStruct((npad, F), jnp.float32),
    )(A, W2, b2.reshape(1, -1), W3)


def _tc3_body(q_ref, r_ref, w1_ref, b1_ref, w2_ref, b2_ref, w3_ref, b3_ref,
              o_ref):
    p = q_ref[...] * r_ref[...]
    z = jnp.dot(p, w1_ref[...], preferred_element_type=jnp.float32)
    z = jnp.maximum(z + b1_ref[...], 0.0)
    z = jnp.dot(z, w2_ref[...], preferred_element_type=jnp.float32)
    z = jnp.maximum(z + b2_ref[...], 0.0)
    z = jnp.dot(z, w3_ref[...], preferred_element_type=jnp.float32)
    o_ref[...] = jax.nn.sigmoid(z + b3_ref[...])


def _tc_mlp(G, PW1, Pb1, PW2, Pb2, PW3p, Pb3p, R=2048):
    t2 = G.shape[0]
    tp = t2 // 2
    nb = tp // R
    wcols = PW3p.shape[1]
    return pl.pallas_call(
        _tc3_body,
        grid=(nb,),
        in_specs=[
            pl.BlockSpec((R, F), lambda i: (i, 0)),
            pl.BlockSpec((R, F), lambda i: (i + nb, 0)),
            pl.BlockSpec(PW1.shape, lambda i: (0, 0)),
            pl.BlockSpec((1, 2 * F), lambda i: (0, 0)),
            pl.BlockSpec(PW2.shape, lambda i: (0, 0)),
            pl.BlockSpec((1, 2 * F), lambda i: (0, 0)),
            pl.BlockSpec(PW3p.shape, lambda i: (0, 0)),
            pl.BlockSpec((1, wcols), lambda i: (0, 0)),
        ],
        out_specs=pl.BlockSpec((R, wcols), lambda i: (i, 0)),
        out_shape=jax.ShapeDtypeStruct((tp, wcols), jnp.float32),
    )(G, G, PW1, Pb1.reshape(1, -1), PW2, Pb2.reshape(1, -1), PW3p,
      Pb3p.reshape(1, -1))


def kernel(x, adj_t, train_edges, W1, b1, W2, b2, W3, b3,
           PW1, Pb1, PW2, Pb2, PW3, Pb3):
    n, d_in = x.shape
    e = adj_t.shape[1]
    t = train_edges.shape[0]
    assert d_in == F
    n_pad = _round_up(n, NS * 8)

    src, dst = adj_t[0], adj_t[1]

    ec = e // 2
    ec_pad = _round_up(ec, NS * CHUNK)
    pe = ec_pad - ec
    src_es = jnp.concatenate(
        [src.reshape(2, ec), jnp.zeros((2, pe), jnp.int32)], axis=1).reshape(-1)
    dst_es = jnp.concatenate(
        [dst.reshape(2, ec), jnp.full((2, pe), n, jnp.int32)], axis=1).reshape(-1)

    e_pad = _round_up(e, NS * CHUNK)
    pf = e_pad - e
    src_p = jnp.concatenate([src, jnp.zeros((pf,), jnp.int32)])
    dst_p = jnp.concatenate([dst, jnp.full((pf,), n, jnp.int32)])
    src_fs = jnp.concatenate([src_p, src_p + n_pad])
    dst_fs = jnp.concatenate([dst_p, dst_p])

    zeros_init = jnp.zeros((2 * n_pad, F), jnp.float32)

    P1 = _sc_scatter(n, n_pad, ec_pad)(x, src_es, dst_es, zeros_init)
    h1 = _tc_layer1(P1.reshape(2, n_pad, F), W1, b1, n)

    A1 = _sc_scatter(2 * n_pad, n_pad, e_pad)(h1.reshape(2 * n_pad, F),
                                              src_fs, dst_fs, zeros_init)
    m3 = _tc_layer23(A1.reshape(2, n_pad, F), W2, b2, W3, n)

    init3 = jnp.concatenate(
        [jnp.broadcast_to(b3, (n_pad, F)), jnp.zeros((n_pad, F), jnp.float32)])
    P3 = _sc_scatter(n, n_pad, ec_pad)(m3, src_es, dst_es, init3)
    P3 = P3.reshape(2, n_pad, F)
    emb = P3[0] + P3[1]

    t_pad = _round_up(t, NC * NS * CHUNK)
    pt = t_pad - t
    se = jnp.concatenate([train_edges[:, 0], jnp.zeros((pt,), jnp.int32)])
    de = jnp.concatenate([train_edges[:, 1], jnp.zeros((pt,), jnp.int32)])
    gidx = jnp.concatenate([se, de])
    G = _sc_gather(n_pad, 2 * t_pad)(emb, gidx)

    PW3p = jnp.pad(PW3, ((0, 0), (0, 8 - PW3.shape[1])))
    Pb3p = jnp.pad(Pb3, (0, 8 - Pb3.shape[0]))
    p = _tc_mlp(G, PW1, Pb1, PW2, Pb2, PW3p, Pb3p)

    return p[:t, :PW3.shape[1]][None]
